# baseline (speedup 1.0000x reference)
import functools

import jax
import jax.numpy as jnp
from jax import lax
from jax.experimental import pallas as pl
from jax.experimental.pallas import tpu as pltpu
from jax.experimental.pallas import tpu_sc as plsc

N_FEAT = 1000000
K = 16
BATCH = 16384
N_FIELDS = 26

NC = 2
NS = 16
NW = NC * NS
ROWS_PER_W = BATCH // NW
BG = 128
NCHUNK = ROWS_PER_W // BG
NBUF = 2
CHUNK_IDX = BG * N_FIELDS
RUN = 4 * N_FIELDS
NRUN = CHUNK_IDX // RUN
NVEC = CHUNK_IDX // 16

RCB = 1024
RSTEPS = -(-N_FEAT // RCB)
PACKED_ROWS = RSTEPS * 128
PACKED_N = PACKED_ROWS * 8


def _repack_body(x_ref, out_ref):
    x = x_ref[...]
    for c in range(8):
        out_ref[:, c * 16:(c + 1) * 16] = jnp.transpose(x[:, c * 128:(c + 1) * 128])


_repack = pl.pallas_call(
    _repack_body,
    grid=(RSTEPS,),
    in_specs=[pl.BlockSpec((16, RCB), lambda g: (0, g))],
    out_specs=pl.BlockSpec((128, 128), lambda g: (g, 0)),
    out_shape=jax.ShapeDtypeStruct((PACKED_ROWS, 128), jnp.float32),
)


def _mf_body(feat_hbm, bias_hbm, x_hbm, out_hbm,
             rows_v, idx_v, idx2_v, bias_v, out_v,
             sem_in0, sem_in1, sem_out0, sem_out1):
    wid = lax.axis_index("s") * NC + lax.axis_index("c")
    wbase = wid * ROWS_PER_W
    sems_in = (sem_in0, sem_in1)
    sems_out = (sem_out0, sem_out1)

    def in_copies(buf):
        cs = []
        for c in range(NRUN):
            sl = pl.ds(c * RUN, RUN)
            cs.append(pltpu.make_async_copy(
                feat_hbm.at[idx2_v.at[buf, sl]], rows_v.at[buf, sl], sems_in[buf]))
            cs.append(pltpu.make_async_copy(
                bias_hbm.at[idx_v.at[buf, sl]], bias_v.at[buf, sl], sems_in[buf]))
        return cs

    def fire(t, buf):
        base = (wbase + t * BG) * N_FIELDS
        pltpu.sync_copy(x_hbm.at[pl.ds(base, CHUNK_IDX)], idx_v.at[buf])

        def xf_body(w, c):
            v = idx_v[buf, pl.ds(w * 16, 16)]
            s = (jnp.bitwise_and(v, -1024)
                 + (jnp.bitwise_and(v, 127) << 3)
                 + jnp.bitwise_and(v >> 7, 7))
            idx2_v[buf, pl.ds(w * 16, 16)] = s
            return c

        lax.fori_loop(0, NVEC, xf_body, 0)
        for c in in_copies(buf):
            c.start()

    def drain(buf):
        for c in in_copies(buf):
            c.wait()

    def out_copy(t, buf):
        base = wbase + t * BG
        return pltpu.make_async_copy(
            out_v.at[buf], out_hbm.at[pl.ds(base, BG)], sems_out[buf])

    lane = lax.iota(jnp.int32, 16)
    lane_lt10 = lane < 10
    bfly = [jnp.reshape(jnp.bitwise_xor(lane, 1 << p), (16, 1)) for p in range(4)]
    _dnums = lax.GatherDimensionNumbers(
        offset_dims=(), collapsed_slice_dims=(0,), start_index_map=(0,))

    def shuffle(x, idx2):
        return lax.gather(x, idx2, _dnums, slice_sizes=(1,),
                          mode=lax.GatherScatterMode.PROMISE_IN_BOUNDS)

    def compute(buf):
        zeros = jnp.zeros((16,), jnp.float32)

        def row_body(r, fmv):
            j = jnp.bitwise_and(r, 15)
            fb = r * N_FIELDS
            v0 = rows_v[buf, fb]
            s = v0
            q = v0 * v0
            for f in range(1, N_FIELDS):
                v = rows_v[buf, fb + f]
                s = s + v
                q = q + v * v
            w1 = bias_v[buf, pl.ds(fb, 16)]
            w2 = bias_v[buf, pl.ds(fb + 16, 16)]
            e = (s * s - q) * 0.5 + w1 + jnp.where(lane_lt10, w2, 0.0)
            for p in range(4):
                e = e + shuffle(e, bfly[p])
            fmv = jnp.where(lane == j, e, fmv)

            @pl.when(j == 15)
            def _():
                out_v[buf, pl.ds(r - 15, 16)] = fmv

            return jnp.where(j == 15, zeros, fmv)

        lax.fori_loop(0, BG, row_body, zeros)

    fire(0, 0)
    fire(1, 1)
    for t in range(NCHUNK):
        buf = t % NBUF
        drain(buf)
        if t >= NBUF:
            out_copy(t - NBUF, buf).wait()
        compute(buf)
        out_copy(t, buf).start()
        nt = t + NBUF
        if nt < NCHUNK:
            fire(nt, buf)
    for t in range(max(NCHUNK - NBUF, 0), NCHUNK):
        out_copy(t, t % NBUF).wait()


_mf_call = functools.partial(
    pl.kernel,
    out_type=jax.ShapeDtypeStruct((BATCH,), jnp.float32),
    mesh=plsc.VectorSubcoreMesh(core_axis_name="c", subcore_axis_name="s"),
    compiler_params=pltpu.CompilerParams(use_tc_tiling_on_sc=False),
    scratch_types=[
        pltpu.VMEM((NBUF, CHUNK_IDX, K), jnp.float32),
        pltpu.VMEM((NBUF, CHUNK_IDX), jnp.int32),
        pltpu.VMEM((NBUF, CHUNK_IDX), jnp.int32),
        pltpu.VMEM((NBUF, CHUNK_IDX + 16), jnp.float32),
        pltpu.VMEM((NBUF, BG), jnp.float32),
        pltpu.SemaphoreType.DMA,
        pltpu.SemaphoreType.DMA,
        pltpu.SemaphoreType.DMA,
        pltpu.SemaphoreType.DMA,
    ],
)(_mf_body)


def kernel(feat_w, bias_feat_w, train_x):
    x_flat = jnp.reshape(train_x, (BATCH * N_FIELDS,))
    packed = _repack(jnp.transpose(feat_w))
    feat_lin = jnp.reshape(packed, (PACKED_N, K))
    bias_flat = jnp.reshape(bias_feat_w, (N_FEAT,))
    return _mf_call(feat_lin, bias_flat, x_flat)


# speedup vs baseline: 3.8015x; 3.8015x over previous
import functools

import jax
import jax.numpy as jnp
from jax import lax
from jax.experimental import pallas as pl
from jax.experimental.pallas import tpu as pltpu
from jax.experimental.pallas import tpu_sc as plsc

N_FEAT = 1000000
K = 16
BATCH = 16384
N_FIELDS = 26

NC = 2
NS = 16
NW = NC * NS
ROWS_PER_W = BATCH // NW
BG = 128
NCHUNK = ROWS_PER_W // BG
NBUF = 2
CHUNK_IDX = BG * N_FIELDS
RUN = 4 * N_FIELDS
NRUN = CHUNK_IDX // RUN
NVEC = CHUNK_IDX // 16

RCB = 16384
MPIECE = RCB // 8
RSTEPS = -(-N_FEAT // RCB)
PACKED_ROWS = RSTEPS * MPIECE
PACKED_N = PACKED_ROWS * 8


def _repack_body(*refs):
    x_refs, out_ref = refs[:8], refs[8]
    xcat = jnp.concatenate([r[...] for r in x_refs], axis=0)
    out_ref[...] = jnp.transpose(xcat)


_MAXBLK = (N_FEAT - 1) // MPIECE

_repack = pl.pallas_call(
    _repack_body,
    grid=(RSTEPS,),
    in_specs=[
        pl.BlockSpec(
            (16, MPIECE),
            functools.partial(
                lambda g, cc: (0, jnp.minimum(g * 8 + cc, _MAXBLK)), cc=cc))
        for cc in range(8)
    ],
    out_specs=pl.BlockSpec((MPIECE, 128), lambda g: (g, 0)),
    out_shape=jax.ShapeDtypeStruct((PACKED_ROWS, 128), jnp.float32),
)


def _mf_body(feat_hbm, bias_hbm, x_hbm, out_hbm,
             rows_v, idx_v, idx2_v, bias_v, out_v,
             sem_in0, sem_in1, sem_out0, sem_out1):
    wid = lax.axis_index("s") * NC + lax.axis_index("c")
    wbase = wid * ROWS_PER_W
    sems_in = (sem_in0, sem_in1)
    sems_out = (sem_out0, sem_out1)

    def in_copies(buf):
        cs = []
        for c in range(NRUN):
            sl = pl.ds(c * RUN, RUN)
            cs.append(pltpu.make_async_copy(
                feat_hbm.at[idx2_v.at[buf, sl]], rows_v.at[buf, sl], sems_in[buf]))
            cs.append(pltpu.make_async_copy(
                bias_hbm.at[idx_v.at[buf, sl]], bias_v.at[buf, sl], sems_in[buf]))
        return cs

    def fire(t, buf):
        base = (wbase + t * BG) * N_FIELDS
        pltpu.sync_copy(x_hbm.at[pl.ds(base, CHUNK_IDX)], idx_v.at[buf])

        def xf_body(w, c):
            v = idx_v[buf, pl.ds(w * 16, 16)]
            s = (jnp.bitwise_and(v, -RCB)
                 + (jnp.bitwise_and(v, MPIECE - 1) << 3)
                 + jnp.bitwise_and(v >> 11, 7))
            idx2_v[buf, pl.ds(w * 16, 16)] = s
            return c

        lax.fori_loop(0, NVEC, xf_body, 0)
        for c in in_copies(buf):
            c.start()

    def drain(buf):
        for c in in_copies(buf):
            c.wait()

    def out_copy(t, buf):
        base = wbase + t * BG
        return pltpu.make_async_copy(
            out_v.at[buf], out_hbm.at[pl.ds(base, BG)], sems_out[buf])

    lane = lax.iota(jnp.int32, 16)
    lane_lt10 = lane < 10
    bfly = [jnp.reshape(jnp.bitwise_xor(lane, 1 << p), (16, 1)) for p in range(4)]
    _dnums = lax.GatherDimensionNumbers(
        offset_dims=(), collapsed_slice_dims=(0,), start_index_map=(0,))

    def shuffle(x, idx2):
        return lax.gather(x, idx2, _dnums, slice_sizes=(1,),
                          mode=lax.GatherScatterMode.PROMISE_IN_BOUNDS)

    def compute(buf):
        zeros = jnp.zeros((16,), jnp.float32)

        def row_body(r, fmv):
            j = jnp.bitwise_and(r, 15)
            fb = r * N_FIELDS
            v0 = rows_v[buf, fb]
            s = v0
            q = v0 * v0
            for f in range(1, N_FIELDS):
                v = rows_v[buf, fb + f]
                s = s + v
                q = q + v * v
            w1 = bias_v[buf, pl.ds(fb, 16)]
            w2 = bias_v[buf, pl.ds(fb + 16, 16)]
            e = (s * s - q) * 0.5 + w1 + jnp.where(lane_lt10, w2, 0.0)
            for p in range(4):
                e = e + shuffle(e, bfly[p])
            fmv = jnp.where(lane == j, e, fmv)

            @pl.when(j == 15)
            def _():
                out_v[buf, pl.ds(r - 15, 16)] = fmv

            return jnp.where(j == 15, zeros, fmv)

        lax.fori_loop(0, BG, row_body, zeros)

    fire(0, 0)
    fire(1, 1)
    for t in range(NCHUNK):
        buf = t % NBUF
        drain(buf)
        if t >= NBUF:
            out_copy(t - NBUF, buf).wait()
        compute(buf)
        out_copy(t, buf).start()
        nt = t + NBUF
        if nt < NCHUNK:
            fire(nt, buf)
    for t in range(max(NCHUNK - NBUF, 0), NCHUNK):
        out_copy(t, t % NBUF).wait()


_mf_call = functools.partial(
    pl.kernel,
    out_type=jax.ShapeDtypeStruct((BATCH,), jnp.float32),
    mesh=plsc.VectorSubcoreMesh(core_axis_name="c", subcore_axis_name="s"),
    compiler_params=pltpu.CompilerParams(use_tc_tiling_on_sc=False),
    scratch_types=[
        pltpu.VMEM((NBUF, CHUNK_IDX, K), jnp.float32),
        pltpu.VMEM((NBUF, CHUNK_IDX), jnp.int32),
        pltpu.VMEM((NBUF, CHUNK_IDX), jnp.int32),
        pltpu.VMEM((NBUF, CHUNK_IDX + 16), jnp.float32),
        pltpu.VMEM((NBUF, BG), jnp.float32),
        pltpu.SemaphoreType.DMA,
        pltpu.SemaphoreType.DMA,
        pltpu.SemaphoreType.DMA,
        pltpu.SemaphoreType.DMA,
    ],
)(_mf_body)


def kernel(feat_w, bias_feat_w, train_x):
    x_flat = jnp.reshape(train_x, (BATCH * N_FIELDS,))
    feat_t = jnp.transpose(feat_w)
    packed = _repack(*([feat_t] * 8))
    feat_lin = jnp.reshape(packed, (PACKED_N, K))
    bias_flat = jnp.reshape(bias_feat_w, (N_FEAT,))
    return _mf_call(feat_lin, bias_flat, x_flat)


# speedup vs baseline: 4.7612x; 1.2524x over previous
import functools

import jax
import jax.numpy as jnp
from jax import lax
from jax.experimental import pallas as pl
from jax.experimental.pallas import tpu as pltpu
from jax.experimental.pallas import tpu_sc as plsc

N_FEAT = 1000000
K = 16
BATCH = 16384
N_FIELDS = 26

NC = 2
NS = 16
NW = NC * NS
ROWS_PER_W = BATCH // NW
BG = 128
NCHUNK = ROWS_PER_W // BG
NBUF = 2
CHUNK_IDX = BG * N_FIELDS
RUN = 4 * N_FIELDS
NRUN = CHUNK_IDX // RUN
NVEC = CHUNK_IDX // 16

RCB = 16384
MPIECE = RCB // 8
RSTEPS = -(-N_FEAT // RCB)
PACKED_ROWS = RSTEPS * MPIECE
PACKED_N = PACKED_ROWS * 8


def _repack_body(*refs):
    x_refs, b_ref, out_ref, bias_out_ref = refs[:8], refs[8], refs[9], refs[10]
    xcat = jnp.concatenate([r[...] for r in x_refs], axis=0)
    out_ref[...] = jnp.transpose(xcat)
    bias_out_ref[...] = b_ref[0, :]


_MAXBLK = (N_FEAT - 1) // MPIECE

_repack = pl.pallas_call(
    _repack_body,
    grid=(RSTEPS,),
    in_specs=[
        pl.BlockSpec(
            (16, MPIECE),
            functools.partial(
                lambda g, cc: (0, jnp.minimum(g * 8 + cc, _MAXBLK)), cc=cc))
        for cc in range(8)
    ] + [pl.BlockSpec((1, RCB), lambda g: (0, g))],
    out_specs=[
        pl.BlockSpec((MPIECE, 128), lambda g: (g, 0)),
        pl.BlockSpec((RCB,), lambda g: (g,)),
    ],
    out_shape=(
        jax.ShapeDtypeStruct((PACKED_ROWS, 128), jnp.float32),
        jax.ShapeDtypeStruct((N_FEAT,), jnp.float32),
    ),
)


def _mf_body(feat_hbm, bias_hbm, x_hbm, out_hbm,
             rows_v, idx_v, idx2_v, bias_v, out_v,
             sem_in0, sem_in1, sem_out0, sem_out1):
    wid = lax.axis_index("s") * NC + lax.axis_index("c")
    wbase = wid * ROWS_PER_W
    sems_in = (sem_in0, sem_in1)
    sems_out = (sem_out0, sem_out1)

    def in_copies(buf):
        cs = []
        for c in range(NRUN):
            sl = pl.ds(c * RUN, RUN)
            cs.append(pltpu.make_async_copy(
                feat_hbm.at[idx2_v.at[buf, sl]], rows_v.at[buf, sl], sems_in[buf]))
            cs.append(pltpu.make_async_copy(
                bias_hbm.at[idx_v.at[buf, sl]], bias_v.at[buf, sl], sems_in[buf]))
        return cs

    def fire(t, buf):
        base = (wbase + t * BG) * N_FIELDS
        pltpu.sync_copy(x_hbm.at[pl.ds(base, CHUNK_IDX)], idx_v.at[buf])

        def xf_body(w, c):
            v = idx_v[buf, pl.ds(w * 16, 16)]
            s = (jnp.bitwise_and(v, -RCB)
                 + (jnp.bitwise_and(v, MPIECE - 1) << 3)
                 + jnp.bitwise_and(v >> 11, 7))
            idx2_v[buf, pl.ds(w * 16, 16)] = s
            return c

        lax.fori_loop(0, NVEC, xf_body, 0)
        for c in in_copies(buf):
            c.start()

    def drain(buf):
        for c in in_copies(buf):
            c.wait()

    def out_copy(t, buf):
        base = wbase + t * BG
        return pltpu.make_async_copy(
            out_v.at[buf], out_hbm.at[pl.ds(base, BG)], sems_out[buf])

    lane = lax.iota(jnp.int32, 16)
    lane_lt10 = lane < 10
    bfly = [jnp.reshape(jnp.bitwise_xor(lane, 1 << p), (16, 1)) for p in range(4)]
    _dnums = lax.GatherDimensionNumbers(
        offset_dims=(), collapsed_slice_dims=(0,), start_index_map=(0,))

    def shuffle(x, idx2):
        return lax.gather(x, idx2, _dnums, slice_sizes=(1,),
                          mode=lax.GatherScatterMode.PROMISE_IN_BOUNDS)

    def compute(buf):
        zeros = jnp.zeros((16,), jnp.float32)

        def row_body(r, fmv):
            j = jnp.bitwise_and(r, 15)
            fb = r * N_FIELDS
            v0 = rows_v[buf, fb]
            s = v0
            q = v0 * v0
            for f in range(1, N_FIELDS):
                v = rows_v[buf, fb + f]
                s = s + v
                q = q + v * v
            w1 = bias_v[buf, pl.ds(fb, 16)]
            w2 = bias_v[buf, pl.ds(fb + 16, 16)]
            e = (s * s - q) * 0.5 + w1 + jnp.where(lane_lt10, w2, 0.0)
            for p in range(4):
                e = e + shuffle(e, bfly[p])
            fmv = jnp.where(lane == j, e, fmv)

            @pl.when(j == 15)
            def _():
                out_v[buf, pl.ds(r - 15, 16)] = fmv

            return jnp.where(j == 15, zeros, fmv)

        lax.fori_loop(0, BG, row_body, zeros)

    fire(0, 0)
    fire(1, 1)
    for t in range(NCHUNK):
        buf = t % NBUF
        drain(buf)
        if t >= NBUF:
            out_copy(t - NBUF, buf).wait()
        compute(buf)
        out_copy(t, buf).start()
        nt = t + NBUF
        if nt < NCHUNK:
            fire(nt, buf)
    for t in range(max(NCHUNK - NBUF, 0), NCHUNK):
        out_copy(t, t % NBUF).wait()


_mf_call = functools.partial(
    pl.kernel,
    out_type=jax.ShapeDtypeStruct((BATCH,), jnp.float32),
    mesh=plsc.VectorSubcoreMesh(core_axis_name="c", subcore_axis_name="s"),
    compiler_params=pltpu.CompilerParams(use_tc_tiling_on_sc=False),
    scratch_types=[
        pltpu.VMEM((NBUF, CHUNK_IDX, K), jnp.float32),
        pltpu.VMEM((NBUF, CHUNK_IDX), jnp.int32),
        pltpu.VMEM((NBUF, CHUNK_IDX), jnp.int32),
        pltpu.VMEM((NBUF, CHUNK_IDX + 16), jnp.float32),
        pltpu.VMEM((NBUF, BG), jnp.float32),
        pltpu.SemaphoreType.DMA,
        pltpu.SemaphoreType.DMA,
        pltpu.SemaphoreType.DMA,
        pltpu.SemaphoreType.DMA,
    ],
)(_mf_body)


def kernel(feat_w, bias_feat_w, train_x):
    x_flat = jnp.reshape(train_x, (BATCH * N_FIELDS,))
    feat_t = jnp.transpose(feat_w)
    packed, bias_flat = _repack(*([feat_t] * 8), jnp.transpose(bias_feat_w))
    feat_lin = jnp.reshape(packed, (PACKED_N, K))
    return _mf_call(feat_lin, bias_flat, x_flat)


# speedup vs baseline: 5.1244x; 1.0763x over previous
import functools

import jax
import jax.numpy as jnp
from jax import lax
from jax.experimental import pallas as pl
from jax.experimental.pallas import tpu as pltpu
from jax.experimental.pallas import tpu_sc as plsc

N_FEAT = 1000000
K = 16
BATCH = 16384
N_FIELDS = 26

NC = 2
NS = 16
NW = NC * NS
ROWS_PER_W = BATCH // NW
BG = 128
NCHUNK = ROWS_PER_W // BG
NBUF = 2
CHUNK_IDX = BG * N_FIELDS
NVEC = CHUNK_IDX // 16
NCHUNKS_ALL = BATCH // BG

RCB = 16384
MPIECE = RCB // 8
RSTEPS = -(-N_FEAT // RCB)
PACKED_ROWS = RSTEPS * MPIECE
PACKED_N = PACKED_ROWS * 8


def _repack_body(*refs):
    x_refs, b_ref, out_ref, bias_out_ref = refs[:8], refs[8], refs[9], refs[10]
    xcat = jnp.concatenate([r[...] for r in x_refs], axis=0)
    out_ref[...] = jnp.transpose(xcat)
    bias_out_ref[...] = b_ref[0, :]


def _xchunks_body(*refs):
    out_ref = refs[8]
    out_ref[...] = jnp.concatenate([r[...] for r in refs[:8]], axis=0)


_xchunks = pl.pallas_call(
    _xchunks_body,
    grid=(NCHUNKS_ALL // 8,),
    in_specs=[
        pl.BlockSpec((N_FIELDS, BG),
                     functools.partial(lambda g, cc: (0, g * 8 + cc), cc=cc))
        for cc in range(8)
    ],
    out_specs=pl.BlockSpec((8 * N_FIELDS, BG), lambda g: (g, 0)),
    out_shape=jax.ShapeDtypeStruct((NCHUNKS_ALL * N_FIELDS, BG), jnp.int32),
)


_MAXBLK = (N_FEAT - 1) // MPIECE

_repack = pl.pallas_call(
    _repack_body,
    grid=(RSTEPS,),
    in_specs=[
        pl.BlockSpec(
            (16, MPIECE),
            functools.partial(
                lambda g, cc: (0, jnp.minimum(g * 8 + cc, _MAXBLK)), cc=cc))
        for cc in range(8)
    ] + [pl.BlockSpec((1, RCB), lambda g: (0, g))],
    out_specs=[
        pl.BlockSpec((MPIECE, 128), lambda g: (g, 0)),
        pl.BlockSpec((RCB,), lambda g: (g,)),
    ],
    out_shape=(
        jax.ShapeDtypeStruct((PACKED_ROWS, 128), jnp.float32),
        jax.ShapeDtypeStruct((N_FEAT,), jnp.float32),
    ),
)


def _mf_body(feat_hbm, bias_hbm, x_hbm, out_hbm,
             rows_v, idx_v, idx2_v, bias_v, out_v,
             sem_in0, sem_in1, sem_out0, sem_out1):
    wid = lax.axis_index("s") * NC + lax.axis_index("c")
    wbase = wid * ROWS_PER_W
    sems_in = (sem_in0, sem_in1)
    sems_out = (sem_out0, sem_out1)

    def in_copies(buf):
        cs = []
        for f in range(N_FIELDS):
            sl = pl.ds(f * BG, BG)
            cs.append(pltpu.make_async_copy(
                feat_hbm.at[idx2_v.at[buf, sl]], rows_v.at[buf, sl], sems_in[buf]))
            cs.append(pltpu.make_async_copy(
                bias_hbm.at[idx_v.at[buf, sl]], bias_v.at[buf, sl], sems_in[buf]))
        return cs

    def fire(t, buf):
        base = (wbase + t * BG) * N_FIELDS
        pltpu.sync_copy(x_hbm.at[pl.ds(base, CHUNK_IDX)], idx_v.at[buf])

        def xf_body(w, c):
            v = idx_v[buf, pl.ds(w * 16, 16)]
            s = (jnp.bitwise_and(v, -RCB)
                 + (jnp.bitwise_and(v, MPIECE - 1) << 3)
                 + jnp.bitwise_and(v >> 11, 7))
            idx2_v[buf, pl.ds(w * 16, 16)] = s
            return c

        lax.fori_loop(0, NVEC, xf_body, 0)
        for c in in_copies(buf):
            c.start()

    def drain(buf):
        for c in in_copies(buf):
            c.wait()

    def out_copy(t, buf):
        base = wbase + t * BG
        return pltpu.make_async_copy(
            out_v.at[buf], out_hbm.at[pl.ds(base, BG)], sems_out[buf])

    lane = lax.iota(jnp.int32, 16)
    bfly = [jnp.reshape(jnp.bitwise_xor(lane, 1 << p), (16, 1)) for p in range(4)]
    _dnums = lax.GatherDimensionNumbers(
        offset_dims=(), collapsed_slice_dims=(0,), start_index_map=(0,))

    def shuffle(x, idx2):
        return lax.gather(x, idx2, _dnums, slice_sizes=(1,),
                          mode=lax.GatherScatterMode.PROMISE_IN_BOUNDS)

    def compute(buf):
        zeros = jnp.zeros((16,), jnp.float32)

        def row_body(r, fmv):
            j = jnp.bitwise_and(r, 15)
            v0 = rows_v[buf, r]
            s = v0
            q = v0 * v0
            for f in range(1, N_FIELDS):
                v = rows_v[buf, f * BG + r]
                s = s + v
                q = q + v * v
            e = s * s - q
            for p in range(4):
                e = e + shuffle(e, bfly[p])
            fmv = jnp.where(lane == j, e, fmv)

            @pl.when(j == 15)
            def _():
                b0 = r - 15
                bacc = bias_v[buf, pl.ds(b0, 16)]
                for f in range(1, N_FIELDS):
                    bacc = bacc + bias_v[buf, pl.ds(f * BG + b0, 16)]
                out_v[buf, pl.ds(b0, 16)] = fmv * 0.5 + bacc

            return jnp.where(j == 15, zeros, fmv)

        lax.fori_loop(0, BG, row_body, zeros)

    fire(0, 0)
    fire(1, 1)
    for t in range(NCHUNK):
        buf = t % NBUF
        drain(buf)
        if t >= NBUF:
            out_copy(t - NBUF, buf).wait()
        compute(buf)
        out_copy(t, buf).start()
        nt = t + NBUF
        if nt < NCHUNK:
            fire(nt, buf)
    for t in range(max(NCHUNK - NBUF, 0), NCHUNK):
        out_copy(t, t % NBUF).wait()


_mf_call = functools.partial(
    pl.kernel,
    out_type=jax.ShapeDtypeStruct((BATCH,), jnp.float32),
    mesh=plsc.VectorSubcoreMesh(core_axis_name="c", subcore_axis_name="s"),
    compiler_params=pltpu.CompilerParams(use_tc_tiling_on_sc=False),
    scratch_types=[
        pltpu.VMEM((NBUF, CHUNK_IDX, K), jnp.float32),
        pltpu.VMEM((NBUF, CHUNK_IDX), jnp.int32),
        pltpu.VMEM((NBUF, CHUNK_IDX), jnp.int32),
        pltpu.VMEM((NBUF, CHUNK_IDX + 16), jnp.float32),
        pltpu.VMEM((NBUF, BG), jnp.float32),
        pltpu.SemaphoreType.DMA,
        pltpu.SemaphoreType.DMA,
        pltpu.SemaphoreType.DMA,
        pltpu.SemaphoreType.DMA,
    ],
)(_mf_body)


def kernel(feat_w, bias_feat_w, train_x):
    x_t = jnp.transpose(train_x)
    x_flat = jnp.reshape(_xchunks(*([x_t] * 8)), (BATCH * N_FIELDS,))
    feat_t = jnp.transpose(feat_w)
    packed, bias_flat = _repack(*([feat_t] * 8), jnp.transpose(bias_feat_w))
    feat_lin = jnp.reshape(packed, (PACKED_N, K))
    return _mf_call(feat_lin, bias_flat, x_flat)


# speedup vs baseline: 5.4074x; 1.0552x over previous
import functools

import jax
import jax.numpy as jnp
from jax import lax
from jax.experimental import pallas as pl
from jax.experimental.pallas import tpu as pltpu
from jax.experimental.pallas import tpu_sc as plsc

N_FEAT = 1000000
K = 16
BATCH = 16384
N_FIELDS = 26

NC = 2
NS = 16
NW = NC * NS
ROWS_PER_W = BATCH // NW
BG = 128
NCHUNK = ROWS_PER_W // BG
NBUF = 2
CHUNK_IDX = BG * N_FIELDS
NVEC = CHUNK_IDX // 16
NCHUNKS_ALL = BATCH // BG

RCB = 16384
MPIECE = RCB // 8
RSTEPS = -(-N_FEAT // RCB)
PACKED_ROWS = RSTEPS * MPIECE
PACKED_N = PACKED_ROWS * 8


def _repack_body(*refs):
    x_refs, out_ref = refs[:8], refs[8]
    xcat = jnp.concatenate([r[...] for r in x_refs], axis=0)
    out_ref[...] = jnp.transpose(xcat)


def _xchunks_body(*refs):
    b_ref, out_ref, bias_out_ref = refs[8], refs[9], refs[10]
    out_ref[...] = jnp.concatenate([r[...] for r in refs[:8]], axis=0)
    bias_out_ref[...] = b_ref[0, :]


_XBIAS_CB = 65536


_xchunks = pl.pallas_call(
    _xchunks_body,
    grid=(NCHUNKS_ALL // 8,),
    in_specs=[
        pl.BlockSpec((N_FIELDS, BG),
                     functools.partial(lambda g, cc: (0, g * 8 + cc), cc=cc))
        for cc in range(8)
    ] + [pl.BlockSpec((1, _XBIAS_CB), lambda g: (0, g))],
    out_specs=[
        pl.BlockSpec((8 * N_FIELDS, BG), lambda g: (g, 0)),
        pl.BlockSpec((_XBIAS_CB,), lambda g: (g,)),
    ],
    out_shape=(
        jax.ShapeDtypeStruct((NCHUNKS_ALL * N_FIELDS, BG), jnp.int32),
        jax.ShapeDtypeStruct((N_FEAT,), jnp.float32),
    ),
)


_MAXBLK = (N_FEAT - 1) // MPIECE

_repack = pl.pallas_call(
    _repack_body,
    grid=(RSTEPS,),
    in_specs=[
        pl.BlockSpec(
            (16, MPIECE),
            functools.partial(
                lambda g, cc: (0, jnp.minimum(g * 8 + cc, _MAXBLK)), cc=cc))
        for cc in range(8)
    ],
    out_specs=pl.BlockSpec((MPIECE, 128), lambda g: (g, 0)),
    out_shape=jax.ShapeDtypeStruct((PACKED_ROWS, 128), jnp.float32),
)


def _bias_body(bias_hbm, x_hbm, idx2_hbm, bsum_hbm,
               idx_v, idx2_v, bias_v, bsum_v,
               sem_in0, sem_in1):
    wid = lax.axis_index("s") * NC + lax.axis_index("c")
    wbase = wid * ROWS_PER_W
    sems_in = (sem_in0, sem_in1)

    def in_copies(buf):
        return [pltpu.make_async_copy(
            bias_hbm.at[idx_v.at[buf, pl.ds(f * BG, BG)]],
            bias_v.at[buf, pl.ds(f * BG, BG)], sems_in[buf])
            for f in range(N_FIELDS)]

    def fire(t, buf):
        base = (wbase + t * BG) * N_FIELDS
        pltpu.sync_copy(x_hbm.at[pl.ds(base, CHUNK_IDX)], idx_v.at[buf])

        def xf_body(w, c):
            v = idx_v[buf, pl.ds(w * 16, 16)]
            s = (jnp.bitwise_and(v, -RCB)
                 + (jnp.bitwise_and(v, MPIECE - 1) << 3)
                 + jnp.bitwise_and(v >> 11, 7))
            idx2_v[buf, pl.ds(w * 16, 16)] = s
            return c

        lax.fori_loop(0, NVEC, xf_body, 0)
        pltpu.sync_copy(idx2_v.at[buf], idx2_hbm.at[pl.ds(base, CHUNK_IDX)])
        for c in in_copies(buf):
            c.start()

    def compute(t, buf):
        def grp_body(g, c):
            b0 = g * 16
            acc = bias_v[buf, pl.ds(b0, 16)]
            for f in range(1, N_FIELDS):
                acc = acc + bias_v[buf, pl.ds(f * BG + b0, 16)]
            bsum_v[buf, pl.ds(b0, 16)] = acc
            return c

        lax.fori_loop(0, BG // 16, grp_body, 0)
        pltpu.sync_copy(bsum_v.at[buf], bsum_hbm.at[pl.ds(wbase + t * BG, BG)])

    fire(0, 0)
    fire(1, 1)
    for t in range(NCHUNK):
        buf = t % NBUF
        for c in in_copies(buf):
            c.wait()
        compute(t, buf)
        nt = t + NBUF
        if nt < NCHUNK:
            fire(nt, buf)


_bias_call = functools.partial(
    pl.kernel,
    out_type=(
        jax.ShapeDtypeStruct((BATCH * N_FIELDS,), jnp.int32),
        jax.ShapeDtypeStruct((BATCH,), jnp.float32),
    ),
    mesh=plsc.VectorSubcoreMesh(core_axis_name="c", subcore_axis_name="s"),
    compiler_params=pltpu.CompilerParams(use_tc_tiling_on_sc=False),
    scratch_types=[
        pltpu.VMEM((NBUF, CHUNK_IDX), jnp.int32),
        pltpu.VMEM((NBUF, CHUNK_IDX), jnp.int32),
        pltpu.VMEM((NBUF, CHUNK_IDX), jnp.float32),
        pltpu.VMEM((NBUF, BG), jnp.float32),
        pltpu.SemaphoreType.DMA,
        pltpu.SemaphoreType.DMA,
    ],
)(_bias_body)


def _mf_body(feat_hbm, idx2_hbm, bsum_hbm, out_hbm,
             rows_v, idx2_v, bsum_v, out_v,
             sem_in0, sem_in1, sem_out0, sem_out1):
    wid = lax.axis_index("s") * NC + lax.axis_index("c")
    wbase = wid * ROWS_PER_W
    sems_in = (sem_in0, sem_in1)
    sems_out = (sem_out0, sem_out1)

    def in_copies(buf):
        return [pltpu.make_async_copy(
            feat_hbm.at[idx2_v.at[buf, pl.ds(f * BG, BG)]],
            rows_v.at[buf, pl.ds(f * BG, BG)], sems_in[buf])
            for f in range(N_FIELDS)]

    def fire(t, buf):
        base = wbase + t * BG
        pltpu.sync_copy(idx2_hbm.at[pl.ds(base * N_FIELDS, CHUNK_IDX)],
                        idx2_v.at[buf])
        pltpu.sync_copy(bsum_hbm.at[pl.ds(base, BG)], bsum_v.at[buf])
        for c in in_copies(buf):
            c.start()

    def drain(buf):
        for c in in_copies(buf):
            c.wait()

    def out_copy(t, buf):
        base = wbase + t * BG
        return pltpu.make_async_copy(
            out_v.at[buf], out_hbm.at[pl.ds(base, BG)], sems_out[buf])

    lane = lax.iota(jnp.int32, 16)
    bfly = [jnp.reshape(jnp.bitwise_xor(lane, 1 << p), (16, 1)) for p in range(4)]
    _dnums = lax.GatherDimensionNumbers(
        offset_dims=(), collapsed_slice_dims=(0,), start_index_map=(0,))

    def shuffle(x, idx2):
        return lax.gather(x, idx2, _dnums, slice_sizes=(1,),
                          mode=lax.GatherScatterMode.PROMISE_IN_BOUNDS)

    def compute(buf):
        zeros = jnp.zeros((16,), jnp.float32)

        def row_body(r, fmv):
            j = jnp.bitwise_and(r, 15)
            v0 = rows_v[buf, r]
            s = v0
            q = v0 * v0
            for f in range(1, N_FIELDS):
                v = rows_v[buf, f * BG + r]
                s = s + v
                q = q + v * v
            e = s * s - q
            for p in range(4):
                e = e + shuffle(e, bfly[p])
            fmv = jnp.where(lane == j, e, fmv)

            @pl.when(j == 15)
            def _():
                b0 = r - 15
                out_v[buf, pl.ds(b0, 16)] = (
                    fmv * 0.5 + bsum_v[buf, pl.ds(b0, 16)])

            return jnp.where(j == 15, zeros, fmv)

        lax.fori_loop(0, BG, row_body, zeros)

    fire(0, 0)
    fire(1, 1)
    for t in range(NCHUNK):
        buf = t % NBUF
        drain(buf)
        if t >= NBUF:
            out_copy(t - NBUF, buf).wait()
        compute(buf)
        out_copy(t, buf).start()
        nt = t + NBUF
        if nt < NCHUNK:
            fire(nt, buf)
    for t in range(max(NCHUNK - NBUF, 0), NCHUNK):
        out_copy(t, t % NBUF).wait()


_mf_call = functools.partial(
    pl.kernel,
    out_type=jax.ShapeDtypeStruct((BATCH,), jnp.float32),
    mesh=plsc.VectorSubcoreMesh(core_axis_name="c", subcore_axis_name="s"),
    compiler_params=pltpu.CompilerParams(use_tc_tiling_on_sc=False),
    scratch_types=[
        pltpu.VMEM((NBUF, CHUNK_IDX, K), jnp.float32),
        pltpu.VMEM((NBUF, CHUNK_IDX), jnp.int32),
        pltpu.VMEM((NBUF, BG), jnp.float32),
        pltpu.VMEM((NBUF, BG), jnp.float32),
        pltpu.SemaphoreType.DMA,
        pltpu.SemaphoreType.DMA,
        pltpu.SemaphoreType.DMA,
        pltpu.SemaphoreType.DMA,
    ],
)(_mf_body)


def kernel(feat_w, bias_feat_w, train_x):
    x_t = jnp.transpose(train_x)
    xc, bias_flat = _xchunks(*([x_t] * 8), jnp.transpose(bias_feat_w))
    x_flat = jnp.reshape(xc, (BATCH * N_FIELDS,))
    feat_t = jnp.transpose(feat_w)
    idx2, bsum = _bias_call(bias_flat, x_flat)
    packed = _repack(*([feat_t] * 8))
    feat_lin = jnp.reshape(packed, (PACKED_N, K))
    return _mf_call(feat_lin, idx2, bsum)
